# baseline (speedup 1.0000x reference)
import functools

import jax
import jax.numpy as jnp
from jax.experimental import pallas as pl
from jax.experimental.pallas import tpu as pltpu

SDIM = 64
VDIM = 16
V3 = 3 * VDIM
MDIM = SDIM + 2 * VDIM
BN = 512


def _silu(x):
    return x * jax.nn.sigmoid(x)


def _node_pre_body(s_ref, v_ref, gamma_ref, beta_ref, w1_ref, w2_ref,
                   sbar_ref, vbar_ref, a_ref, b2_ref):
    s = s_ref[...]
    mu = jnp.mean(s, axis=-1, keepdims=True)
    var = jnp.mean((s - mu) * (s - mu), axis=-1, keepdims=True)
    sbar = (s - mu) * jax.lax.rsqrt(var + 1e-5) * gamma_ref[...] + beta_ref[...]
    sbar_ref[...] = sbar
    v = v_ref[...]
    inv = jax.lax.rsqrt(jnp.sum(v * v, axis=-1, keepdims=True) / VDIM + 1e-5)
    vbar_ref[...] = v * inv
    a_ref[...] = jnp.dot(sbar, w1_ref[...], preferred_element_type=jnp.float32)
    b2_ref[...] = jnp.dot(sbar, w2_ref[...], preferred_element_type=jnp.float32)


def _node_pre(s, vflat, gamma_i, beta_i, w1, w2):
    n = s.shape[0]
    grid = (pl.cdiv(n, BN),)
    out_shapes = (
        jax.ShapeDtypeStruct((n, SDIM), jnp.float32),
        jax.ShapeDtypeStruct((n, V3), jnp.float32),
        jax.ShapeDtypeStruct((n, MDIM), jnp.float32),
        jax.ShapeDtypeStruct((n, MDIM), jnp.float32),
    )
    return pl.pallas_call(
        _node_pre_body,
        grid=grid,
        in_specs=[
            pl.BlockSpec((BN, SDIM), lambda i: (i, 0)),
            pl.BlockSpec((BN, V3), lambda i: (i, 0)),
            pl.BlockSpec((1, SDIM), lambda i: (0, 0)),
            pl.BlockSpec((1, SDIM), lambda i: (0, 0)),
            pl.BlockSpec((SDIM, MDIM), lambda i: (0, 0)),
            pl.BlockSpec((SDIM, MDIM), lambda i: (0, 0)),
        ],
        out_specs=(
            pl.BlockSpec((BN, SDIM), lambda i: (i, 0)),
            pl.BlockSpec((BN, V3), lambda i: (i, 0)),
            pl.BlockSpec((BN, MDIM), lambda i: (i, 0)),
            pl.BlockSpec((BN, MDIM), lambda i: (i, 0)),
        ),
        out_shape=out_shapes,
    )(s, vflat, gamma_i.reshape(1, SDIM), beta_i.reshape(1, SDIM), w1, w2)


def _node_post_body(sbar_ref, vbar_ref, sagg_ref, vagg_ref, deg_ref,
                    wu_ref, bu_ref, wv_ref, s_out_ref, v_out_ref, *, last):
    sbar = sbar_ref[...]
    wu = wu_ref[...]
    upd = (jnp.dot(sbar, wu[:SDIM], preferred_element_type=jnp.float32)
           + jnp.dot(sagg_ref[...], wu[SDIM:], preferred_element_type=jnp.float32)
           + bu_ref[...])
    if not last:
        upd = _silu(upd)
    s_out_ref[...] = sbar + upd
    vagg = vagg_ref[...] / deg_ref[...]
    wv = wv_ref[...]
    parts = [jnp.dot(vagg[:, c * VDIM:(c + 1) * VDIM], wv,
                     preferred_element_type=jnp.float32) for c in range(3)]
    v_out_ref[...] = vbar_ref[...] + jnp.concatenate(parts, axis=1)


def _node_post(sbar, vbar, s_agg, v_agg, deg, wu, bu, wv, last):
    n = sbar.shape[0]
    grid = (pl.cdiv(n, BN),)
    out_shapes = (
        jax.ShapeDtypeStruct((n, SDIM), jnp.float32),
        jax.ShapeDtypeStruct((n, V3), jnp.float32),
    )
    return pl.pallas_call(
        functools.partial(_node_post_body, last=last),
        grid=grid,
        in_specs=[
            pl.BlockSpec((BN, SDIM), lambda i: (i, 0)),
            pl.BlockSpec((BN, V3), lambda i: (i, 0)),
            pl.BlockSpec((BN, SDIM), lambda i: (i, 0)),
            pl.BlockSpec((BN, V3), lambda i: (i, 0)),
            pl.BlockSpec((BN, 1), lambda i: (i, 0)),
            pl.BlockSpec((2 * SDIM, SDIM), lambda i: (0, 0)),
            pl.BlockSpec((1, SDIM), lambda i: (0, 0)),
            pl.BlockSpec((VDIM, VDIM), lambda i: (0, 0)),
        ],
        out_specs=(
            pl.BlockSpec((BN, SDIM), lambda i: (i, 0)),
            pl.BlockSpec((BN, V3), lambda i: (i, 0)),
        ),
        out_shape=out_shapes,
    )(sbar, vbar, s_agg, v_agg, deg, wu, bu.reshape(1, SDIM), wv)


def _edge_stage(a_t, b2_t, c_t, w4, r, d, src, dst, edge_attr, vbar, n, first):
    msum = a_t[src] + b2_t[dst] + c_t[edge_attr] + d[:, None] * w4
    m = _silu(msum)
    ms = m[:, :SDIM]
    gv = m[:, SDIM:SDIM + VDIM]
    gr = m[:, SDIM + VDIM:]
    mv = r[:, :, None] * gr[:, None, :]
    if not first:
        e = src.shape[0]
        mv = mv + vbar.reshape(-1, 3, VDIM)[src] * gv[:, None, :]
    s_agg = jax.ops.segment_sum(ms, dst, num_segments=n)
    v_agg = jax.ops.segment_sum(mv.reshape(-1, V3), dst, num_segments=n)
    return s_agg, v_agg


def kernel(x, t, pos, edge_index, edge_attr, batch, atom_emb, bond_emb,
           W_t1, b_t1, W_t2, b_t2, W_msg, b_msg, W_upd, b_upd, W_v,
           gamma, beta, W_down, b_down):
    n = x.shape[0]
    L = W_msg.shape[0]
    perm = jnp.argsort(edge_index[1])
    src = edge_index[0][perm]
    dst = edge_index[1][perm]
    edge_attr = edge_attr[perm]

    rvec = pos[dst] - pos[src]
    d = jnp.sqrt(jnp.maximum(jnp.sum(rvec * rvec, axis=-1), 1e-6))
    rvec = rvec / d[:, None]

    temb = _silu(_silu(t @ W_t1 + b_t1) @ W_t2 + b_t2)
    s = atom_emb[x] + temb[batch]
    vflat = jnp.zeros((n, V3), dtype=jnp.float32)

    deg = jnp.maximum(
        jax.ops.segment_sum(jnp.ones_like(d), dst, num_segments=n), 1.0)
    deg2 = deg[:, None]

    for i in range(L):
        w1 = W_msg[i, :SDIM]
        w2 = W_msg[i, SDIM:2 * SDIM]
        w3 = W_msg[i, 2 * SDIM:2 * SDIM + 16]
        w4 = W_msg[i, 2 * SDIM + 16]
        c_t = bond_emb @ w3 + b_msg[i]
        sbar, vbar, a_t, b2_t = _node_pre(s, vflat, gamma[i], beta[i], w1, w2)
        s_agg, v_agg = _edge_stage(a_t, b2_t, c_t, w4, rvec, d, src, dst,
                                   edge_attr, vbar, n, first=(i == 0))
        s, vflat = _node_post(sbar, vbar, s_agg, v_agg, deg2,
                              W_upd[i], b_upd[i], W_v[i], last=(i == L - 1))

    v = vflat.reshape(n, 3, VDIM)
    out = (v @ W_down + b_down).squeeze(-1)
    return out


# speedup vs baseline: 3.2743x; 3.2743x over previous
import functools

import jax
import jax.numpy as jnp
from jax import lax
from jax.experimental import pallas as pl
from jax.experimental.pallas import tpu as pltpu
from jax.experimental.pallas import tpu_sc as plsc

SDIM = 64
VDIM = 16
V3 = 3 * VDIM
MDIM = SDIM + 2 * VDIM
BN = 512

NPAD = 50176
RANGES = 8
RANGE_N = NPAD // RANGES
ACC_ROWS = 6400
DUMP = RANGE_N
EB = 128
GW = 128
NTILES = 16


def _silu(x):
    return x * jax.nn.sigmoid(x)


def _node_pre_body(s_ref, v_ref, gamma_ref, beta_ref, w1_ref, w2_ref,
                   sbar_ref, vbar_ref, a_ref, b2_ref, vp_ref):
    s = s_ref[...]
    mu = jnp.mean(s, axis=-1, keepdims=True)
    var = jnp.mean((s - mu) * (s - mu), axis=-1, keepdims=True)
    sbar = (s - mu) * jax.lax.rsqrt(var + 1e-5) * gamma_ref[...] + beta_ref[...]
    sbar_ref[...] = sbar
    v = v_ref[...]
    inv = jax.lax.rsqrt(jnp.sum(v * v, axis=-1, keepdims=True) / VDIM + 1e-5)
    vbar_ref[...] = v * inv
    z = jnp.zeros((sbar.shape[0], GW - MDIM), jnp.float32)
    zv = jnp.zeros((sbar.shape[0], GW - V3), jnp.float32)
    a_ref[...] = jnp.concatenate(
        [jnp.dot(sbar, w1_ref[...], preferred_element_type=jnp.float32), z], axis=1)
    b2_ref[...] = jnp.concatenate(
        [jnp.dot(sbar, w2_ref[...], preferred_element_type=jnp.float32), z], axis=1)
    vp_ref[...] = jnp.concatenate([v * inv, zv], axis=1)


def _node_pre(s, vflat, gamma_i, beta_i, w1, w2):
    n = s.shape[0]
    grid = (pl.cdiv(n, BN),)
    out_shapes = (
        jax.ShapeDtypeStruct((n, SDIM), jnp.float32),
        jax.ShapeDtypeStruct((n, V3), jnp.float32),
        jax.ShapeDtypeStruct((n, GW), jnp.float32),
        jax.ShapeDtypeStruct((n, GW), jnp.float32),
        jax.ShapeDtypeStruct((n, GW), jnp.float32),
    )
    return pl.pallas_call(
        _node_pre_body,
        grid=grid,
        in_specs=[
            pl.BlockSpec((BN, SDIM), lambda i: (i, 0)),
            pl.BlockSpec((BN, V3), lambda i: (i, 0)),
            pl.BlockSpec((1, SDIM), lambda i: (0, 0)),
            pl.BlockSpec((1, SDIM), lambda i: (0, 0)),
            pl.BlockSpec((SDIM, MDIM), lambda i: (0, 0)),
            pl.BlockSpec((SDIM, MDIM), lambda i: (0, 0)),
        ],
        out_specs=(
            pl.BlockSpec((BN, SDIM), lambda i: (i, 0)),
            pl.BlockSpec((BN, V3), lambda i: (i, 0)),
            pl.BlockSpec((BN, GW), lambda i: (i, 0)),
            pl.BlockSpec((BN, GW), lambda i: (i, 0)),
            pl.BlockSpec((BN, GW), lambda i: (i, 0)),
        ),
        out_shape=out_shapes,
    )(s, vflat, gamma_i.reshape(1, SDIM), beta_i.reshape(1, SDIM), w1, w2)


def _node_post_body(sbar_ref, vbar_ref, agg_ref, deg_ref,
                    wu_ref, bu_ref, wv_ref, s_out_ref, v_out_ref, *, last):
    sbar = sbar_ref[...]
    wu = wu_ref[...]
    aggv = agg_ref[...]
    upd = (jnp.dot(sbar, wu[:SDIM], preferred_element_type=jnp.float32)
           + jnp.dot(aggv[:, :SDIM], wu[SDIM:], preferred_element_type=jnp.float32)
           + bu_ref[...])
    if not last:
        upd = _silu(upd)
    s_out_ref[...] = sbar + upd
    vagg = aggv[:, SDIM:SDIM + V3] / deg_ref[...]
    wv = wv_ref[...]
    parts = [jnp.dot(vagg[:, c * VDIM:(c + 1) * VDIM], wv,
                     preferred_element_type=jnp.float32) for c in range(3)]
    v_out_ref[...] = vbar_ref[...] + jnp.concatenate(parts, axis=1)


def _node_post(sbar, vbar, agg, deg, wu, bu, wv, last):
    n = sbar.shape[0]
    grid = (pl.cdiv(n, BN),)
    out_shapes = (
        jax.ShapeDtypeStruct((n, SDIM), jnp.float32),
        jax.ShapeDtypeStruct((n, V3), jnp.float32),
    )
    return pl.pallas_call(
        functools.partial(_node_post_body, last=last),
        grid=grid,
        in_specs=[
            pl.BlockSpec((BN, SDIM), lambda i: (i, 0)),
            pl.BlockSpec((BN, V3), lambda i: (i, 0)),
            pl.BlockSpec((BN, GW), lambda i: (i, 0)),
            pl.BlockSpec((BN, 1), lambda i: (i, 0)),
            pl.BlockSpec((2 * SDIM, SDIM), lambda i: (0, 0)),
            pl.BlockSpec((1, SDIM), lambda i: (0, 0)),
            pl.BlockSpec((VDIM, VDIM), lambda i: (0, 0)),
        ],
        out_specs=(
            pl.BlockSpec((BN, SDIM), lambda i: (i, 0)),
            pl.BlockSpec((BN, V3), lambda i: (i, 0)),
        ),
        out_shape=out_shapes,
    )(sbar, vbar, agg, deg, wu, bu.reshape(1, SDIM), wv)


def _sc_edge_body(first,
                  a_hbm, b_hbm, vb_hbm, c_hbm, w4_hbm, eoff_hbm,
                  src_hbm, dst_hbm, ea_hbm, r0_hbm, r1_hbm, r2_hbm, d_hbm,
                  agg_hbm,
                  src_st, dst_st, ea_st, r0_st, r1_st, r2_st, d_st,
                  gidx, didx, sidx, a_buf, b_buf, vs_buf, upd,
                  c_buf, w4_buf, eoff_buf, acc, sem):
    c = lax.axis_index("c")
    s = lax.axis_index("s")
    zeros16 = jnp.zeros((16,), jnp.float32)
    lane = lax.iota(jnp.int32, 16)

    def _bcast(ref, i):
        return plsc.load_gather(ref, [jnp.full((16,), i, jnp.int32)])

    pltpu.sync_copy(c_hbm, c_buf)
    pltpu.sync_copy(w4_hbm, w4_buf)
    pltpu.sync_copy(eoff_hbm, eoff_buf)
    w4v = [w4_buf[pl.ds(16 * k, 16)] for k in range(6)]

    for p in range(RANGES // 2):
        r_id = (RANGES // 2) * c + p
        e_lo = _bcast(eoff_buf, r_id)[0]
        e_hi = _bcast(eoff_buf, r_id + 1)[0]
        range_base = r_id * RANGE_N

        def _zero_row(e, _):
            for k in range(8):
                upd[e, pl.ds(16 * k, 16)] = zeros16
            return 0
        lax.fori_loop(0, EB, _zero_row, 0)
        rows_per_tile = ACC_ROWS // NTILES
        zbase = s * rows_per_tile
        for k in range(rows_per_tile // EB):
            pltpu.sync_copy(upd, acc.at[pl.ds(zbase + k * EB, EB)])
        rem = rows_per_tile % EB
        if rem:
            pltpu.sync_copy(upd.at[pl.ds(0, rem)],
                            acc.at[pl.ds(zbase + (rows_per_tile - rem), rem)])
        plsc.subcore_barrier()

        start_al = e_lo & ~7
        total = e_hi - start_al
        chunk = ((total + NTILES - 1) // NTILES + 7) & ~7
        my0 = start_al + s * chunk
        my1 = jnp.minimum(my0 + chunk, e_hi)
        nb = jnp.maximum(my1 - my0 + EB - 1, 0) // EB

        def _batch(b, _):
            base = pl.multiple_of(my0 + b * EB, 8)
            hs = []
            for hbm, st in ((src_hbm, src_st), (dst_hbm, dst_st),
                            (ea_hbm, ea_st), (r0_hbm, r0_st),
                            (r1_hbm, r1_st), (r2_hbm, r2_st),
                            (d_hbm, d_st)):
                hs.append(pltpu.async_copy(hbm.at[pl.ds(base, EB)], st, sem))
            for h in hs:
                h.wait()

            def _prep(g, _):
                eid = base + 16 * g + lane
                valid = (eid >= e_lo) & (eid < my1)
                srcv = src_st[pl.ds(16 * g, 16)]
                dstv = dst_st[pl.ds(16 * g, 16)]
                gi = jnp.where(valid, srcv, 0)
                di = jnp.where(valid, dstv, 0)
                si = jnp.where(valid, dstv - range_base, DUMP + lane)
                gidx[g // 8, pl.ds((g % 8) * 16, 16)] = gi
                didx[g // 8, pl.ds((g % 8) * 16, 16)] = di
                sidx[g // 8, pl.ds((g % 8) * 16, 16)] = si
                return 0
            lax.fori_loop(0, EB // 16, _prep, 0)

            hs = []
            for j in range(EB // 128):
                hs.append(pltpu.async_copy(
                    a_hbm.at[gidx.at[j]],
                    a_buf.at[pl.ds(j * 128, 128)], sem))
                hs.append(pltpu.async_copy(
                    b_hbm.at[didx.at[j]],
                    b_buf.at[pl.ds(j * 128, 128)], sem))
                if not first:
                    hs.append(pltpu.async_copy(
                        vb_hbm.at[gidx.at[j]],
                        vs_buf.at[pl.ds(j * 128, 128)], sem))
            for h in hs:
                h.wait()

            def _edge(e, _):
                dv = _bcast(d_st, e)
                eac = jnp.minimum(jnp.maximum(_bcast(ea_st, e)[0], 0), 7) * MDIM
                m = []
                for k in range(6):
                    xv = (a_buf[e, pl.ds(16 * k, 16)]
                          + b_buf[e, pl.ds(16 * k, 16)]
                          + c_buf[pl.ds(eac + 16 * k, 16)]
                          + dv * w4v[k])
                    sg = 1.0 / (1.0 + jnp.exp(-xv))
                    m.append(xv * sg)
                for k in range(4):
                    upd[e, pl.ds(16 * k, 16)] = m[k]
                gv = m[4]
                gr = m[5]
                for c3, rst in enumerate((r0_st, r1_st, r2_st)):
                    mv = _bcast(rst, e) * gr
                    if not first:
                        mv = mv + vs_buf[e, pl.ds(16 * c3, 16)] * gv
                    upd[e, pl.ds(SDIM + 16 * c3, 16)] = mv
                return 0
            lax.fori_loop(0, EB, _edge, 0)

            for j in range(EB // 128):
                pltpu.sync_copy(upd.at[pl.ds(j * 128, 128)],
                                acc.at[sidx.at[j]], add=True)
            return 0

        lax.fori_loop(0, nb, _batch, 0)
        plsc.subcore_barrier()

        dpt = RANGE_N // NTILES
        pltpu.sync_copy(acc.at[pl.ds(s * dpt, dpt)],
                        agg_hbm.at[pl.ds(range_base + s * dpt, dpt)])
        plsc.subcore_barrier()


def _sc_edge_stage(first, a_t, b2_t, vbar, c_flat, w4, eoff,
                   src_s, dst_s, ea_s, r0, r1, r2, d_s):
    mesh = plsc.VectorSubcoreMesh(core_axis_name="c", subcore_axis_name="s",
                                  num_cores=2, num_subcores=NTILES)
    f = pl.kernel(
        functools.partial(_sc_edge_body, first),
        out_type=jax.ShapeDtypeStruct((NPAD, GW), jnp.float32),
        mesh=mesh,
        scratch_types=[
            pltpu.VMEM((EB,), jnp.int32),
            pltpu.VMEM((EB,), jnp.int32),
            pltpu.VMEM((EB,), jnp.int32),
            pltpu.VMEM((EB,), jnp.float32),
            pltpu.VMEM((EB,), jnp.float32),
            pltpu.VMEM((EB,), jnp.float32),
            pltpu.VMEM((EB,), jnp.float32),
            pltpu.VMEM((EB // 128, 128), jnp.int32),
            pltpu.VMEM((EB // 128, 128), jnp.int32),
            pltpu.VMEM((EB // 128, 128), jnp.int32),
            pltpu.VMEM((EB, GW), jnp.float32),
            pltpu.VMEM((EB, GW), jnp.float32),
            pltpu.VMEM((EB, GW), jnp.float32),
            pltpu.VMEM((EB, GW), jnp.float32),
            pltpu.VMEM((MDIM * 8,), jnp.float32),
            pltpu.VMEM((MDIM,), jnp.float32),
            pltpu.VMEM((16,), jnp.int32),
            pltpu.VMEM_SHARED((ACC_ROWS, GW), jnp.float32),
            pltpu.SemaphoreType.DMA,
        ],
        compiler_params=pltpu.CompilerParams(needs_layout_passes=False),
    )
    return f(a_t, b2_t, vbar, c_flat, w4, eoff,
             src_s, dst_s, ea_s, r0, r1, r2, d_s)


def _edge_stage(a_t, b2_t, c_t, w4, r, d, src, dst, edge_attr, vbar, n, first):
    msum = a_t[src] + b2_t[dst] + c_t[edge_attr] + d[:, None] * w4
    m = _silu(msum)
    ms = m[:, :SDIM]
    gv = m[:, SDIM:SDIM + VDIM]
    gr = m[:, SDIM + VDIM:]
    mv = r[:, :, None] * gr[:, None, :]
    if not first:
        e = src.shape[0]
        mv = mv + vbar.reshape(-1, 3, VDIM)[src] * gv[:, None, :]
    s_agg = jax.ops.segment_sum(ms, dst, num_segments=n)
    v_agg = jax.ops.segment_sum(mv.reshape(-1, V3), dst, num_segments=n)
    return s_agg, v_agg


def kernel(x, t, pos, edge_index, edge_attr, batch, atom_emb, bond_emb,
           W_t1, b_t1, W_t2, b_t2, W_msg, b_msg, W_upd, b_upd, W_v,
           gamma, beta, W_down, b_down):
    n = x.shape[0]
    L = W_msg.shape[0]
    e = edge_index.shape[1]
    perm = jnp.argsort(edge_index[1])
    src = edge_index[0][perm]
    dst = edge_index[1][perm]
    ea_s = edge_attr[perm]

    rvec = pos[dst] - pos[src]
    d = jnp.sqrt(jnp.maximum(jnp.sum(rvec * rvec, axis=-1), 1e-6))
    rvec = rvec / d[:, None]

    epad = 2 * EB
    src_s = jnp.pad(src, (0, epad))
    dst_p = jnp.pad(dst, (0, epad))
    ea_p = jnp.pad(ea_s, (0, epad))
    r0 = jnp.pad(rvec[:, 0], (0, epad))
    r1 = jnp.pad(rvec[:, 1], (0, epad))
    r2 = jnp.pad(rvec[:, 2], (0, epad))
    d_p = jnp.pad(d, (0, epad))

    bounds = jnp.arange(0, NPAD + 1, RANGE_N, dtype=jnp.int32)
    eoff = jnp.searchsorted(dst, bounds).astype(jnp.int32)
    eoff = jnp.pad(eoff, (0, 16 - eoff.shape[0]))

    off_full = jnp.searchsorted(dst, jnp.arange(n + 1, dtype=jnp.int32))
    deg = jnp.maximum((off_full[1:] - off_full[:-1]).astype(jnp.float32), 1.0)
    deg2 = jnp.pad(deg, (0, NPAD - n), constant_values=1.0)[:, None]

    temb = _silu(_silu(t @ W_t1 + b_t1) @ W_t2 + b_t2)
    s = atom_emb[x] + temb[batch]
    s = jnp.pad(s, ((0, NPAD - n), (0, 0)))
    vflat = jnp.zeros((NPAD, V3), dtype=jnp.float32)

    for i in range(L):
        w1 = W_msg[i, :SDIM]
        w2 = W_msg[i, SDIM:2 * SDIM]
        w3 = W_msg[i, 2 * SDIM:2 * SDIM + 16]
        w4 = W_msg[i, 2 * SDIM + 16]
        c_flat = (bond_emb @ w3 + b_msg[i]).reshape(-1)
        sbar, vbar, a_t, b2_t, vp_t = _node_pre(s, vflat, gamma[i], beta[i],
                                                w1, w2)
        agg = _sc_edge_stage(i == 0, a_t, b2_t, vp_t, c_flat, w4,
                             eoff, src_s, dst_p, ea_p, r0, r1, r2, d_p)
        s, vflat = _node_post(sbar, vbar, agg, deg2,
                              W_upd[i], b_upd[i], W_v[i], last=(i == L - 1))

    v = vflat[:n].reshape(n, 3, VDIM)
    out = (v @ W_down + b_down).squeeze(-1)
    return out


# speedup vs baseline: 4.7869x; 1.4620x over previous
import functools

import jax
import jax.numpy as jnp
from jax import lax
from jax.experimental import pallas as pl
from jax.experimental.pallas import tpu as pltpu
from jax.experimental.pallas import tpu_sc as plsc

SDIM = 64
VDIM = 16
V3 = 3 * VDIM
MDIM = SDIM + 2 * VDIM
BN = 512

NPAD = 50176
RANGES = 16
RANGE_N = NPAD // RANGES
ACC_ROWS = 3200
DUMP = RANGE_N
EB = 96
GW = 128
NTILES = 16
EDW = 8


def _silu(x):
    return x * jax.nn.sigmoid(x)


def _seed_body(x_ref, b_ref, t_ref, wt1_ref, bt1_ref, wt2_ref, bt2_ref,
               ae_ref, s0_ref):
    temb = _silu(_silu(t_ref[...] @ wt1_ref[...] + bt1_ref[...])
                 @ wt2_ref[...] + bt2_ref[...])
    xv = x_ref[0, 0, :]
    bv = b_ref[0, 0, :]
    oh_x = (xv[:, None] == lax.broadcasted_iota(jnp.int32, (1, 16), 1)
            ).astype(jnp.float32)
    oh_b = (bv[:, None] == lax.broadcasted_iota(jnp.int32, (1, 64), 1)
            ).astype(jnp.float32)
    s0_ref[...] = (jnp.dot(oh_x, ae_ref[...], preferred_element_type=jnp.float32)
                   + jnp.dot(oh_b, temb, preferred_element_type=jnp.float32))


def _seed(x_p, batch_p, t, W_t1, b_t1, W_t2, b_t2, atom_emb):
    nb = NPAD // BN
    return pl.pallas_call(
        _seed_body,
        grid=(nb,),
        in_specs=[
            pl.BlockSpec((1, 1, BN), lambda i: (i, 0, 0)),
            pl.BlockSpec((1, 1, BN), lambda i: (i, 0, 0)),
            pl.BlockSpec(t.shape, lambda i: (0, 0)),
            pl.BlockSpec(W_t1.shape, lambda i: (0, 0)),
            pl.BlockSpec((1, SDIM), lambda i: (0, 0)),
            pl.BlockSpec(W_t2.shape, lambda i: (0, 0)),
            pl.BlockSpec((1, SDIM), lambda i: (0, 0)),
            pl.BlockSpec((16, SDIM), lambda i: (0, 0)),
        ],
        out_specs=pl.BlockSpec((BN, SDIM), lambda i: (i, 0)),
        out_shape=jax.ShapeDtypeStruct((NPAD, SDIM), jnp.float32),
    )(x_p, batch_p, t, W_t1, b_t1.reshape(1, SDIM), W_t2,
      b_t2.reshape(1, SDIM), atom_emb)


def _node_pre_body(s_ref, v_ref, gamma_ref, beta_ref, w1_ref, w2_ref,
                   sbar_ref, vbar_ref, a_ref, b2_ref, vp_ref):
    s = s_ref[...]
    mu = jnp.mean(s, axis=-1, keepdims=True)
    var = jnp.mean((s - mu) * (s - mu), axis=-1, keepdims=True)
    sbar = (s - mu) * jax.lax.rsqrt(var + 1e-5) * gamma_ref[...] + beta_ref[...]
    sbar_ref[...] = sbar
    v = v_ref[...]
    inv = jax.lax.rsqrt(jnp.sum(v * v, axis=-1, keepdims=True) / VDIM + 1e-5)
    vbar_ref[...] = v * inv
    z = jnp.zeros((sbar.shape[0], GW - MDIM), jnp.float32)
    zv = jnp.zeros((sbar.shape[0], GW - V3), jnp.float32)
    a_ref[...] = jnp.concatenate(
        [jnp.dot(sbar, w1_ref[...], preferred_element_type=jnp.float32), z],
        axis=1)
    b2_ref[...] = jnp.concatenate(
        [jnp.dot(sbar, w2_ref[...], preferred_element_type=jnp.float32), z],
        axis=1)
    vp_ref[...] = jnp.concatenate([v * inv, zv], axis=1)


def _node_pre(s, vflat, gamma_i, beta_i, w1, w2):
    n = s.shape[0]
    grid = (pl.cdiv(n, BN),)
    out_shapes = (
        jax.ShapeDtypeStruct((n, SDIM), jnp.float32),
        jax.ShapeDtypeStruct((n, V3), jnp.float32),
        jax.ShapeDtypeStruct((n, GW), jnp.float32),
        jax.ShapeDtypeStruct((n, GW), jnp.float32),
        jax.ShapeDtypeStruct((n, GW), jnp.float32),
    )
    return pl.pallas_call(
        _node_pre_body,
        grid=grid,
        in_specs=[
            pl.BlockSpec((BN, SDIM), lambda i: (i, 0)),
            pl.BlockSpec((BN, V3), lambda i: (i, 0)),
            pl.BlockSpec((1, SDIM), lambda i: (0, 0)),
            pl.BlockSpec((1, SDIM), lambda i: (0, 0)),
            pl.BlockSpec((SDIM, MDIM), lambda i: (0, 0)),
            pl.BlockSpec((SDIM, MDIM), lambda i: (0, 0)),
        ],
        out_specs=(
            pl.BlockSpec((BN, SDIM), lambda i: (i, 0)),
            pl.BlockSpec((BN, V3), lambda i: (i, 0)),
            pl.BlockSpec((BN, GW), lambda i: (i, 0)),
            pl.BlockSpec((BN, GW), lambda i: (i, 0)),
            pl.BlockSpec((BN, GW), lambda i: (i, 0)),
        ),
        out_shape=out_shapes,
    )(s, vflat, gamma_i.reshape(1, SDIM), beta_i.reshape(1, SDIM), w1, w2)


def _node_post_body(sbar_ref, vbar_ref, agg_ref, wu_ref, bu_ref, wv_ref,
                    s_out_ref, v_out_ref, *, last):
    sbar = sbar_ref[...]
    wu = wu_ref[...]
    aggv = agg_ref[...]
    upd = (jnp.dot(sbar, wu[:SDIM], preferred_element_type=jnp.float32)
           + jnp.dot(aggv[:, :SDIM], wu[SDIM:],
                     preferred_element_type=jnp.float32)
           + bu_ref[...])
    if not last:
        upd = _silu(upd)
    s_out_ref[...] = sbar + upd
    deg = jnp.maximum(aggv[:, SDIM + V3:SDIM + V3 + 1], 1.0)
    vagg = aggv[:, SDIM:SDIM + V3] / deg
    wv = wv_ref[...]
    parts = [jnp.dot(vagg[:, c * VDIM:(c + 1) * VDIM], wv,
                     preferred_element_type=jnp.float32) for c in range(3)]
    v_out_ref[...] = vbar_ref[...] + jnp.concatenate(parts, axis=1)


def _node_post(sbar, vbar, agg, wu, bu, wv, last):
    n = sbar.shape[0]
    grid = (pl.cdiv(n, BN),)
    out_shapes = (
        jax.ShapeDtypeStruct((n, SDIM), jnp.float32),
        jax.ShapeDtypeStruct((n, V3), jnp.float32),
    )
    return pl.pallas_call(
        functools.partial(_node_post_body, last=last),
        grid=grid,
        in_specs=[
            pl.BlockSpec((BN, SDIM), lambda i: (i, 0)),
            pl.BlockSpec((BN, V3), lambda i: (i, 0)),
            pl.BlockSpec((BN, GW), lambda i: (i, 0)),
            pl.BlockSpec((2 * SDIM, SDIM), lambda i: (0, 0)),
            pl.BlockSpec((1, SDIM), lambda i: (0, 0)),
            pl.BlockSpec((VDIM, VDIM), lambda i: (0, 0)),
        ],
        out_specs=(
            pl.BlockSpec((BN, SDIM), lambda i: (i, 0)),
            pl.BlockSpec((BN, V3), lambda i: (i, 0)),
        ),
        out_shape=out_shapes,
    )(sbar, vbar, agg, wu, bu.reshape(1, SDIM), wv)


def _sc_edge_body(first,
                  a_hbm, b_hbm, vb_hbm, c_hbm, w4_hbm, eoff_hbm, ed_hbm,
                  agg_hbm,
                  ed_st, gidx, didx, sidx, a_buf, b_buf, vs_buf, upd,
                  c_buf, w4_buf, eoff_buf, acc, sems):
    c = lax.axis_index("c")
    s = lax.axis_index("s")
    zeros16 = jnp.zeros((16,), jnp.float32)
    lane = lax.iota(jnp.int32, 16)
    one0 = jnp.where(lane == 0, 1.0, 0.0).astype(jnp.float32)

    def _gat(ref, idxv):
        return plsc.load_gather(ref, [idxv])

    def _bcast(ref, i):
        return plsc.load_gather(ref, [jnp.full((16,), i, jnp.int32)])

    pltpu.sync_copy(c_hbm, c_buf)
    pltpu.sync_copy(w4_hbm, w4_buf)
    pltpu.sync_copy(eoff_hbm, eoff_buf)
    w4v = [w4_buf[pl.ds(16 * k, 16)] for k in range(6)]

    for p in range(RANGES // 2):
        r_id = (RANGES // 2) * c + p
        e_lo = _bcast(eoff_buf, r_id)[0]
        e_hi = _bcast(eoff_buf, r_id + 1)[0]
        range_base = r_id * RANGE_N

        def _zero_row(e, _):
            for k in range(8):
                upd[e, pl.ds(16 * k, 16)] = zeros16
            return 0
        lax.fori_loop(0, EB, _zero_row, 0)
        rows_per_tile = ACC_ROWS // NTILES
        zbase = s * rows_per_tile
        for k in range(rows_per_tile // EB):
            pltpu.sync_copy(upd, acc.at[pl.ds(zbase + k * EB, EB)])
        rem = rows_per_tile % EB
        if rem:
            pltpu.sync_copy(upd.at[pl.ds(0, rem)],
                            acc.at[pl.ds(zbase + (rows_per_tile - rem), rem)])
        plsc.subcore_barrier()

        start_al = e_lo & ~7
        total = e_hi - start_al
        chunk = ((total + NTILES - 1) // NTILES + 7) & ~7
        my0 = start_al + s * chunk
        my1 = jnp.minimum(my0 + chunk, e_hi)
        nb = jnp.maximum(my1 - my0 + EB - 1, 0) // EB

        def _stage_prep_issue(t, buf):
            base = pl.multiple_of(my0 + t * EB, 8)
            pltpu.sync_copy(ed_hbm.at[pl.ds(base * EDW, EB * EDW)],
                            ed_st[buf])
            edr = ed_st[buf]

            def _prep(g, _):
                eid = base + 16 * g + lane
                valid = (eid >= e_lo) & (eid < my1)
                eoffs = (16 * g + lane) * EDW
                srcv = plsc.bitcast(_gat(edr, eoffs), jnp.int32)
                dstv = plsc.bitcast(_gat(edr, eoffs + 1), jnp.int32)
                gidx[buf][pl.ds(16 * g, 16)] = jnp.where(valid, srcv, 0)
                didx[buf][pl.ds(16 * g, 16)] = jnp.where(valid, dstv, 0)
                sidx[buf][pl.ds(16 * g, 16)] = jnp.where(
                    valid, dstv - range_base, DUMP + lane)
                return 0
            lax.fori_loop(0, EB // 16, _prep, 0)
            pltpu.async_copy(a_hbm.at[gidx[buf]], a_buf[buf], sems[buf])
            pltpu.async_copy(b_hbm.at[didx[buf]], b_buf[buf], sems[buf])
            if not first:
                pltpu.async_copy(vb_hbm.at[gidx[buf]], vs_buf[buf], sems[buf])

        def _wait(buf):
            pltpu.make_async_copy(a_hbm.at[gidx[buf]], a_buf[buf],
                                  sems[buf]).wait()
            pltpu.make_async_copy(b_hbm.at[didx[buf]], b_buf[buf],
                                  sems[buf]).wait()
            if not first:
                pltpu.make_async_copy(vb_hbm.at[gidx[buf]], vs_buf[buf],
                                      sems[buf]).wait()

        def _compute_scatter(buf):
            edr = ed_st[buf]

            def _edge(e, _):
                dv = _bcast(edr, e * EDW + 6)
                eac = jnp.minimum(jnp.maximum(
                    plsc.bitcast(_bcast(edr, e * EDW + 2), jnp.int32)[0],
                    0), 7) * MDIM
                m = []
                for k in range(6):
                    xv = (a_buf[buf][e, pl.ds(16 * k, 16)]
                          + b_buf[buf][e, pl.ds(16 * k, 16)]
                          + c_buf[pl.ds(eac + 16 * k, 16)]
                          + dv * w4v[k])
                    sg = 1.0 / (1.0 + jnp.exp(-xv))
                    m.append(xv * sg)
                for k in range(4):
                    upd[e, pl.ds(16 * k, 16)] = m[k]
                gv = m[4]
                gr = m[5]
                for c3 in range(3):
                    mv = _bcast(edr, e * EDW + 3 + c3) * gr
                    if not first:
                        mv = mv + vs_buf[buf][e, pl.ds(16 * c3, 16)] * gv
                    upd[e, pl.ds(SDIM + 16 * c3, 16)] = mv
                upd[e, pl.ds(SDIM + V3, 16)] = one0
                return 0
            lax.fori_loop(0, EB, _edge, 0)
            pltpu.sync_copy(upd, acc.at[sidx[buf]], add=True)

        @pl.when(nb > 0)
        def _prologue():
            _stage_prep_issue(0, 0)

        def _pair(h, _):
            t = 2 * h

            @pl.when(t + 1 < nb)
            def _issue_b():
                _stage_prep_issue(t + 1, 1)

            _wait(0)
            _compute_scatter(0)

            @pl.when(t + 2 < nb)
            def _issue_a():
                _stage_prep_issue(t + 2, 0)

            @pl.when(t + 1 < nb)
            def _consume_b():
                _wait(1)
                _compute_scatter(1)
            return 0

        lax.fori_loop(0, (nb + 1) // 2, _pair, 0)
        plsc.subcore_barrier()

        dpt = RANGE_N // 8

        @pl.when(s < 8)
        def _drain():
            pltpu.sync_copy(acc.at[pl.ds(s * dpt, dpt)],
                            agg_hbm.at[pl.ds(range_base + s * dpt, dpt)])
        plsc.subcore_barrier()


def _sc_edge_stage(first, a_t, b2_t, vp_t, c_flat, w4, eoff, edata):
    mesh = plsc.VectorSubcoreMesh(core_axis_name="c", subcore_axis_name="s",
                                  num_cores=2, num_subcores=NTILES)
    f = pl.kernel(
        functools.partial(_sc_edge_body, first),
        out_type=jax.ShapeDtypeStruct((NPAD, GW), jnp.float32),
        mesh=mesh,
        scratch_types=[
            (pltpu.VMEM((EB * EDW,), jnp.float32),) * 2,
            (pltpu.VMEM((EB,), jnp.int32),) * 2,
            (pltpu.VMEM((EB,), jnp.int32),) * 2,
            (pltpu.VMEM((EB,), jnp.int32),) * 2,
            (pltpu.VMEM((EB, GW), jnp.float32),) * 2,
            (pltpu.VMEM((EB, GW), jnp.float32),) * 2,
            (pltpu.VMEM((EB, GW), jnp.float32),) * 2,
            pltpu.VMEM((EB, GW), jnp.float32),
            pltpu.VMEM((MDIM * 8,), jnp.float32),
            pltpu.VMEM((MDIM,), jnp.float32),
            pltpu.VMEM((32,), jnp.int32),
            pltpu.VMEM_SHARED((ACC_ROWS, GW), jnp.float32),
            (pltpu.SemaphoreType.DMA, pltpu.SemaphoreType.DMA),
        ],
        compiler_params=pltpu.CompilerParams(needs_layout_passes=False),
    )
    return f(a_t, b2_t, vp_t, c_flat, w4, eoff, edata)


def kernel(x, t, pos, edge_index, edge_attr, batch, atom_emb, bond_emb,
           W_t1, b_t1, W_t2, b_t2, W_msg, b_msg, W_upd, b_upd, W_v,
           gamma, beta, W_down, b_down):
    n = x.shape[0]
    L = W_msg.shape[0]
    perm = jnp.argsort(edge_index[1])
    src = edge_index[0][perm]
    dst = edge_index[1][perm]
    ea_s = edge_attr[perm]

    rvec = pos[dst] - pos[src]
    d = jnp.sqrt(jnp.maximum(jnp.sum(rvec * rvec, axis=-1), 1e-6))
    rvec = rvec / d[:, None]

    bc = lambda a: lax.bitcast_convert_type(a.astype(jnp.int32), jnp.float32)
    edata = jnp.stack([bc(src), bc(dst), bc(ea_s), rvec[:, 0], rvec[:, 1],
                       rvec[:, 2], d, jnp.zeros_like(d)], axis=1)
    edata = jnp.pad(edata, ((0, 2 * EB + 16), (0, 0))).reshape(-1)

    bounds = jnp.arange(0, NPAD + 1, RANGE_N, dtype=jnp.int32)
    eoff = jnp.searchsorted(dst, bounds).astype(jnp.int32)
    eoff = jnp.pad(eoff, (0, 32 - eoff.shape[0]))

    x_p = jnp.pad(x, (0, NPAD - n)).reshape(NPAD // BN, 1, BN)
    batch_p = jnp.pad(batch, (0, NPAD - n)).reshape(NPAD // BN, 1, BN)
    s = _seed(x_p, batch_p, t, W_t1, b_t1, W_t2, b_t2, atom_emb)
    vflat = jnp.zeros((NPAD, V3), dtype=jnp.float32)

    for i in range(L):
        w1 = W_msg[i, :SDIM]
        w2 = W_msg[i, SDIM:2 * SDIM]
        w3 = W_msg[i, 2 * SDIM:2 * SDIM + 16]
        w4 = W_msg[i, 2 * SDIM + 16]
        c_flat = (bond_emb @ w3 + b_msg[i]).reshape(-1)
        sbar, vbar, a_t, b2_t, vp_t = _node_pre(s, vflat, gamma[i], beta[i],
                                                w1, w2)
        agg = _sc_edge_stage(i == 0, a_t, b2_t, vp_t, c_flat, w4, eoff, edata)
        s, vflat = _node_post(sbar, vbar, agg,
                              W_upd[i], b_upd[i], W_v[i], last=(i == L - 1))

    v = vflat[:n].reshape(n, 3, VDIM)
    out = (v @ W_down + b_down).squeeze(-1)
    return out
